# baseline (speedup 1.0000x reference)
import functools

import jax
import jax.numpy as jnp
from jax import lax
from jax.experimental import pallas as pl
from jax.experimental.pallas import tpu as pltpu
from jax.experimental.pallas import tpu_sc as plsc

_LANES = 16
_ROWS = 128


@functools.lru_cache(maxsize=None)
def _build_lookup(N, F, V, D):
    info = plsc.get_sparse_core_info()
    NC, NS = info.num_cores, info.num_subcores
    NW = NC * NS
    assert N % (NW * _ROWS) == 0
    chunk = N // NW
    n_batches = chunk // _ROWS
    mesh = plsc.VectorSubcoreMesh(core_axis_name="c", subcore_axis_name="s")

    @functools.partial(
        pl.kernel,
        mesh=mesh,
        out_type=jax.ShapeDtypeStruct((N, D), jnp.float32),
        scratch_types=[
            pltpu.VMEM((chunk,), jnp.int32),
            pltpu.VMEM((_ROWS, D), jnp.float32),
            pltpu.SemaphoreType.DMA,
        ],
        compiler_params=pltpu.CompilerParams(use_tc_tiling_on_sc=False),
    )
    def lookup(x_hbm, tab_hbm, out_hbm, idx_v, rows_v, gsem):
        wid = lax.axis_index("s") * NC + lax.axis_index("c")
        base = wid * chunk
        pltpu.sync_copy(x_hbm.at[pl.ds(base, chunk)], idx_v)

        def add_offsets(i, carry):
            sl = idx_v[pl.ds(i * _LANES, _LANES)]
            pos = base + i * _LANES + lax.iota(jnp.int32, _LANES)
            idx_v[pl.ds(i * _LANES, _LANES)] = sl + (pos % F) * V
            return carry

        lax.fori_loop(0, chunk // _LANES, add_offsets, 0)

        def gather_batch(j, carry):
            cp = pltpu.async_copy(
                tab_hbm.at[idx_v.at[pl.ds(j * _ROWS, _ROWS)]], rows_v, gsem
            )
            cp.wait()
            pltpu.sync_copy(rows_v, out_hbm.at[pl.ds(base + j * _ROWS, _ROWS)])
            return carry

        lax.fori_loop(0, n_batches, gather_batch, 0)

    return lookup


def kernel(x, tables):
    B, F = x.shape
    Ft, V, D = tables.shape
    N = B * F
    x_flat = x.reshape(N)
    tab_flat = tables.reshape(Ft * V, D)
    out_flat = _build_lookup(N, F, V, D)(x_flat, tab_flat)
    return out_flat.reshape(B, F, D)


# speedup vs baseline: 1.0526x; 1.0526x over previous
import functools

import jax
import jax.numpy as jnp
from jax import lax
from jax.experimental import pallas as pl
from jax.experimental.pallas import tpu as pltpu
from jax.experimental.pallas import tpu_sc as plsc

_LANES = 16
_ROWS = 128
_NBUF = 8
_DEPTH = 4


@functools.lru_cache(maxsize=None)
def _build_lookup(N, F, V, D):
    info = plsc.get_sparse_core_info()
    NC, NS = info.num_cores, info.num_subcores
    NW = NC * NS
    assert N % (NW * _ROWS) == 0
    chunk = N // NW
    n_batches = chunk // _ROWS
    assert n_batches % _NBUF == 0 and n_batches >= 2 * _NBUF
    mesh = plsc.VectorSubcoreMesh(core_axis_name="c", subcore_axis_name="s")

    @functools.partial(
        pl.kernel,
        mesh=mesh,
        out_type=jax.ShapeDtypeStruct((N, D), jnp.float32),
        scratch_types=[
            pltpu.VMEM((chunk,), jnp.int32),
            pltpu.VMEM((_NBUF, _ROWS, D), jnp.float32),
            pltpu.SemaphoreType.DMA((_NBUF,)),
            pltpu.SemaphoreType.DMA((_NBUF,)),
        ],
        compiler_params=pltpu.CompilerParams(use_tc_tiling_on_sc=False),
    )
    def lookup(x_hbm, tab_hbm, out_hbm, idx_v, rows_v, gsem, osem):
        wid = lax.axis_index("s") * NC + lax.axis_index("c")
        base = wid * chunk
        pltpu.sync_copy(x_hbm.at[pl.ds(base, chunk)], idx_v)

        def add_offsets(i, carry):
            sl = idx_v[pl.ds(i * _LANES, _LANES)]
            pos = base + i * _LANES + lax.iota(jnp.int32, _LANES)
            idx_v[pl.ds(i * _LANES, _LANES)] = sl + (pos % F) * V
            return carry

        lax.fori_loop(0, chunk // _LANES, add_offsets, 0)

        def gather(j, b):
            pltpu.async_copy(
                tab_hbm.at[idx_v.at[pl.ds(j * _ROWS, _ROWS)]],
                rows_v.at[b],
                gsem.at[b],
            )

        def wait_gather(b):
            pltpu.make_async_copy(
                tab_hbm.at[pl.ds(0, _ROWS)], rows_v.at[b], gsem.at[b]
            ).wait()

        def copy_out(j, b):
            pltpu.async_copy(
                rows_v.at[b], out_hbm.at[pl.ds(base + j * _ROWS, _ROWS)],
                osem.at[b],
            )

        def wait_copy_out(b):
            pltpu.make_async_copy(
                rows_v.at[b], out_hbm.at[pl.ds(base, _ROWS)], osem.at[b]
            ).wait()

        for b in range(_DEPTH):
            gather(b, b)

        def outer(g, carry):
            for b in range(_NBUF):
                j = g * _NBUF + b
                wait_gather(b)
                copy_out(j, b)
                j2 = j + _DEPTH
                b2 = (b + _DEPTH) % _NBUF

                @pl.when(j2 < n_batches)
                def _():
                    @pl.when(j2 >= _NBUF)
                    def _():
                        wait_copy_out(b2)

                    gather(j2, b2)

            return carry

        lax.fori_loop(0, n_batches // _NBUF, outer, 0)

        for b in range(_NBUF):
            wait_copy_out(b)

    return lookup


def kernel(x, tables):
    B, F = x.shape
    Ft, V, D = tables.shape
    N = B * F
    x_flat = x.reshape(N)
    tab_flat = tables.reshape(Ft * V, D)
    out_flat = _build_lookup(N, F, V, D)(x_flat, tab_flat)
    return out_flat.reshape(B, F, D)
